# baseline (speedup 1.0000x reference)
import math

import jax
import jax.numpy as jnp
from jax.experimental import pallas as pl
from jax.experimental.pallas import tpu as pltpu

_B, _S, _D, _H, _HD, _L, _E, _K, _F = 8, 64, 128, 8, 16, 64, 16, 2, 512
_SD = _S * _D
_LD = _L * _D
_T = _B * _S
_RP = _T * _K
_MT = 64
_RPP = _RP + 2 * _MT
_NBLK = 256
_EPB = 16


def _dot_t(a, b):
    return jax.lax.dot_general(a, b, (((1,), (1,)), ((), ())),
                               preferred_element_type=jnp.float32)


def _dot(a, b):
    return jax.lax.dot_general(a, b, (((1,), (0,)), ((), ())),
                               preferred_element_type=jnp.float32)


def _latent_kernel(xf_ref, rmsw_ref, w_ref, b_ref, o_ref, h_ref):
    @pl.when(pl.program_id(0) == 0)
    def _():
        f32 = jnp.float32
        gi = jax.lax.broadcasted_iota(jnp.int32, (_SD, _S), 0)
        gj = jax.lax.broadcasted_iota(jnp.int32, (_SD, _S), 1)
        blk = (gi // _D == gj).astype(f32)
        ei = jax.lax.broadcasted_iota(jnp.int32, (_S, _SD), 0)
        ej = jax.lax.broadcasted_iota(jnp.int32, (_S, _SD), 1)
        expand = (ei == ej // _D).astype(f32)
        xf = xf_ref[...]
        ms = _dot(xf * xf, blk) * (1.0 / _D)
        scale = _dot(jax.lax.rsqrt(ms + 1e-5), expand)
        h_ref[...] = xf * scale * rmsw_ref[...]

    o_ref[...] = _dot_t(h_ref[...], w_ref[...]) + b_ref[...]


def _attn_route(h_ref, lat_ref, x_ref, wq_ref, bq_ref, wk_ref, bk_ref,
                wv_ref, bv_ref, wo_ref, bo_ref, rms2_ref, rw_ref, rb_ref,
                o_ref, sh2_scr, ggt_scr, offs_vscr):
    f32 = jnp.float32
    pos = jax.lax.broadcasted_iota(jnp.int32, (_S, _D), 0).astype(f32)
    lane = jax.lax.broadcasted_iota(jnp.int32, (_S, _D), 1)
    pair = (lane % _HD) // 2
    inv_freq = jnp.exp(pair.astype(f32) * (-2.0 / _HD) * math.log(10000.0))
    ang = pos * inv_freq
    rc = jnp.cos(ang)
    rs = jnp.sin(ang)
    ssign = jnp.where(lane % 2 == 0, -rs, rs)
    ai = jax.lax.broadcasted_iota(jnp.int32, (_D, _D), 0)
    bi = jax.lax.broadcasted_iota(jnp.int32, (_D, _D), 1)
    perm = ((ai // 2 == bi // 2) & (ai != bi)).astype(f32)
    eye = (ai == bi).astype(f32)
    scale = 1.0 / math.sqrt(_HD)
    _HL = _H * _L
    kr = jax.lax.broadcasted_iota(jnp.int32, (_D, _HL), 0)
    kc = jax.lax.broadcasted_iota(jnp.int32, (_D, _HL), 1)
    kmask = (kr // _HD == kc // _L).astype(f32)
    vr = jax.lax.broadcasted_iota(jnp.int32, (_HL, _D), 0)
    vc = jax.lax.broadcasted_iota(jnp.int32, (_HL, _D), 1)
    vmask = (vr // _L == vc // _HD).astype(f32)
    sr = jax.lax.broadcasted_iota(jnp.int32, (_HL, _H), 0)
    sc2 = jax.lax.broadcasted_iota(jnp.int32, (_HL, _H), 1)
    colblk = (sr // _L == sc2).astype(f32)
    er = jax.lax.broadcasted_iota(jnp.int32, (_H, _HL), 0)
    ec = jax.lax.broadcasted_iota(jnp.int32, (_H, _HL), 1)
    rowblk = (er == ec // _L).astype(f32)

    h2s = []
    for b in range(_B):
        h = h_ref[b]
        lat = lat_ref[b]
        q = _dot_t(h, wq_ref[...]) + bq_ref[...]
        k = _dot_t(lat, wk_ref[...]) + bk_ref[...]
        v = _dot_t(lat, wv_ref[...]) + bv_ref[...]
        qr = q * rc + _dot(q, perm) * ssign
        kt = _dot_t(eye, k)
        kexp = jnp.concatenate([kt] * _H, axis=1) * kmask
        sc = _dot(qr, kexp) * scale
        mx = jnp.max(sc, axis=1, keepdims=True)
        p = jnp.exp(sc - mx)
        denom = _dot(_dot(p, colblk), rowblk)
        p = p / denom
        vexp = jnp.concatenate([v] * _H, axis=0) * vmask
        ao = _dot(p, vexp)
        attn_out = _dot_t(ao, wo_ref[...]) + bo_ref[...]
        xr = x_ref[b] + attn_out
        o_ref[pl.ds(b * _S, _S), :] = xr
        ms = jnp.mean(xr * xr, axis=1, keepdims=True)
        h2s.append(xr * jax.lax.rsqrt(ms + 1e-5) * rms2_ref[...])

    h2 = jnp.concatenate(h2s, axis=0)
    logits = _dot_t(h2, rw_ref[...]) + rb_ref[...]
    mx = jnp.max(logits, axis=1, keepdims=True)
    p = jnp.exp(logits - mx)
    probs = p / jnp.sum(p, axis=1, keepdims=True)
    iota_e = jax.lax.broadcasted_iota(jnp.int32, (_T, _E), 1)
    m1 = jnp.max(probs, axis=1, keepdims=True)
    i1 = jnp.min(jnp.where(probs == m1, iota_e, _E), axis=1, keepdims=True)
    sel1 = (iota_e == i1).astype(f32)
    pmask = jnp.where(iota_e == i1, -1.0, probs)
    m2 = jnp.max(pmask, axis=1, keepdims=True)
    i2 = jnp.min(jnp.where(pmask == m2, iota_e, _E), axis=1, keepdims=True)
    sel2 = (iota_e == i2).astype(f32)
    mask = sel1 + sel2

    ti = jax.lax.broadcasted_iota(jnp.int32, (_T, _T), 0)
    tj = jax.lax.broadcasted_iota(jnp.int32, (_T, _T), 1)
    ltri = (tj < ti).astype(f32)
    prefix = _dot(ltri, mask)
    a16 = jax.lax.broadcasted_iota(jnp.int32, (_E, 32), 0)
    b32 = jax.lax.broadcasted_iota(jnp.int32, (_E, 32), 1)
    u32 = (a16 < b32).astype(f32)
    offs32 = _dot(jnp.ones((1, _T), f32), _dot(mask, u32))
    offs_vscr[...] = offs32.astype(jnp.int32)
    offs = offs32[:, :_E]
    posm = offs + prefix
    p0 = jnp.sum(sel1 * posm, axis=1, keepdims=True)
    p1 = jnp.sum(sel2 * posm, axis=1, keepdims=True)
    one = jnp.ones((1, 1), f32)
    p0t = _dot_t(one, p0)
    p1t = _dot_t(one, p1)

    rif = jax.lax.broadcasted_iota(jnp.int32, (_RP, _T), 0).astype(f32)
    gmat = (rif == p0t).astype(f32) + (rif == p1t).astype(f32)
    sh2_scr[pl.ds(0, _RP), :] = _dot(gmat, h2)
    sh2_scr[pl.ds(_RP, _RPP - _RP), :] = jnp.zeros((_RPP - _RP, _D), f32)

    cif = jax.lax.broadcasted_iota(jnp.int32, (_T, _RP), 1).astype(f32)
    ggt_scr[...] = (m1 * (cif == p0).astype(f32)
                    + m2 * (cif == p1).astype(f32))


def _block_kernel(h_ref, lat_ref, x_ref, wq_ref, bq_ref, wk_ref, bk_ref,
                  wv_ref, bv_ref, wo_ref, bo_ref, rms2_ref, rw_ref, rb_ref,
                  w1_ref, b1_ref, ws_ref, bs_ref, w2_ref, b2_ref, o_ref,
                  sh2_scr, ggt_scr, ysc, offs_vscr, offs_sscr, sem):
    g = pl.program_id(0)

    @pl.when(g == 0)
    def _():
        _attn_route(h_ref, lat_ref, x_ref, wq_ref, bq_ref, wk_ref, bk_ref,
                    wv_ref, bv_ref, wo_ref, bo_ref, rms2_ref, rw_ref, rb_ref,
                    o_ref, sh2_scr, ggt_scr, offs_vscr)
        cp = pltpu.make_async_copy(offs_vscr, offs_sscr, sem)
        cp.start()
        cp.wait()
        ysc[...] = jnp.zeros((_RPP, _D), jnp.float32)

    @pl.when((g >= 1) & (g <= _E // _EPB))
    def _():
        bf16 = jnp.bfloat16
        for j in range(_EPB):
            e = (g - 1) * _EPB + j
            start = offs_sscr[0, e]
            end = offs_sscr[0, e + 1]
            base = (start // _MT) * _MT
            w1b = w1_ref[j].astype(bf16)
            wsb = ws_ref[j].astype(bf16)
            w2b = w2_ref[j].astype(bf16)

            def tile(st, w1b=w1b, wsb=wsb, w2b=w2b, j=j, start=start, end=end):
                rows = sh2_scr[pl.ds(st, _MT), :]
                h1 = _dot_t(rows.astype(bf16), w1b) + b1_ref[j]
                hs = jnp.maximum(_dot_t(h1.astype(bf16), wsb) + bs_ref[j],
                                 0.0)
                y = _dot_t(hs.astype(bf16), w2b) + b2_ref[j]
                rid = st + jax.lax.broadcasted_iota(jnp.int32, (_MT, _D), 0)
                valid = (rid >= start) & (rid < end)
                old = ysc[pl.ds(st, _MT), :]
                ysc[pl.ds(st, _MT), :] = jnp.where(valid, y, old)

            tile(base)
            tile(base + _MT)
            nt = (end - base + _MT - 1) // _MT

            def body(t, carry, tile=tile, base=base):
                tile(base + t * _MT)
                return carry

            jax.lax.fori_loop(2, nt, body, 0)

    @pl.when(g == _E // _EPB + 1)
    def _():
        o_ref[...] += _dot(ggt_scr[...], ysc[pl.ds(0, _RP), :])


def kernel(x, rms1_w, rms2_w, latent_W, latent_b, Wq, bq, Wk, bk, Wv, bv,
           Wo, bo, router_W, router_b, e1_W, e1_b, sw_W, sw_b, e2_W, e2_b):
    f32 = jnp.float32

    n_lat = _LD // _NBLK
    rmsw_flat = jnp.broadcast_to(rms1_w, (_S, _D)).reshape(1, _SD)
    latent, h = pl.pallas_call(
        _latent_kernel,
        grid=(n_lat,),
        in_specs=[
            pl.BlockSpec((_B, _SD), lambda i: (0, 0)),
            pl.BlockSpec((1, _SD), lambda i: (0, 0)),
            pl.BlockSpec((_NBLK, _SD), lambda i: (i, 0)),
            pl.BlockSpec((1, _NBLK), lambda i: (0, i)),
        ],
        out_specs=[
            pl.BlockSpec((_B, _NBLK), lambda i: (0, i)),
            pl.BlockSpec((_B, _SD), lambda i: (0, 0)),
        ],
        out_shape=[
            jax.ShapeDtypeStruct((_B, _LD), f32),
            jax.ShapeDtypeStruct((_B, _SD), f32),
        ],
    )(x.reshape(_B, _SD), rmsw_flat, latent_W, latent_b.reshape(1, _LD))

    row = lambda a: a.reshape(1, -1)
    cst = lambda shp: pl.BlockSpec(shp, lambda g: tuple(0 for _ in shp))
    wsel = lambda shp: pl.BlockSpec(
        shp, lambda g: (jnp.clip(g - 1, 0, _E // _EPB - 1),)
        + (0,) * (len(shp) - 1))
    out = pl.pallas_call(
        _block_kernel,
        grid=(_E // _EPB + 2,),
        in_specs=[
            cst((_B, _S, _D)), cst((_B, _L, _D)), cst((_B, _S, _D)),
            cst((_D, _D)), cst((1, _D)),
            cst((_D, _D)), cst((1, _D)),
            cst((_D, _D)), cst((1, _D)),
            cst((_D, _D)), cst((1, _D)),
            cst((1, _D)),
            cst((_E, _D)), cst((1, _E)),
            wsel((_EPB, _F, _D)), wsel((_EPB, 1, _F)),
            wsel((_EPB, _F, _F)), wsel((_EPB, 1, _F)),
            wsel((_EPB, _D, _F)), wsel((_EPB, 1, _D)),
        ],
        out_specs=pl.BlockSpec((_T, _D), lambda g: (0, 0)),
        out_shape=jax.ShapeDtypeStruct((_T, _D), f32),
        scratch_shapes=[
            pltpu.VMEM((_RPP, _D), f32),
            pltpu.VMEM((_T, _RP), f32),
            pltpu.VMEM((_RPP, _D), f32),
            pltpu.VMEM((1, 32), jnp.int32),
            pltpu.SMEM((1, 32), jnp.int32),
            pltpu.SemaphoreType.DMA,
        ],
    )(h.reshape(_B, _S, _D), latent.reshape(_B, _L, _D), x,
      Wq, row(bq), Wk, row(bk), Wv, row(bv), Wo, row(bo),
      row(rms2_w), router_W, row(router_b),
      e1_W, e1_b.reshape(_E, 1, _F), sw_W, sw_b.reshape(_E, 1, _F),
      e2_W, e2_b.reshape(_E, 1, _D))

    return out.reshape(_B, _S, _D)


# speedup vs baseline: 1.0517x; 1.0517x over previous
import math

import jax
import jax.numpy as jnp
from jax.experimental import pallas as pl
from jax.experimental.pallas import tpu as pltpu

_B, _S, _D, _H, _HD, _L, _E, _K, _F = 8, 64, 128, 8, 16, 64, 16, 2, 512
_SD = _S * _D
_LD = _L * _D
_T = _B * _S
_RP = _T * _K
_MT = 64
_RPP = _RP + 2 * _MT
_NBLK = 256
_EPB = 4


def _dot_t(a, b):
    return jax.lax.dot_general(a, b, (((1,), (1,)), ((), ())),
                               preferred_element_type=jnp.float32)


def _dot(a, b):
    return jax.lax.dot_general(a, b, (((1,), (0,)), ((), ())),
                               preferred_element_type=jnp.float32)


def _latent_kernel(xf_ref, rmsw_ref, w_ref, b_ref, o_ref, h_ref):
    @pl.when(pl.program_id(0) == 0)
    def _():
        f32 = jnp.float32
        gi = jax.lax.broadcasted_iota(jnp.int32, (_SD, _S), 0)
        gj = jax.lax.broadcasted_iota(jnp.int32, (_SD, _S), 1)
        blk = (gi // _D == gj).astype(f32)
        ei = jax.lax.broadcasted_iota(jnp.int32, (_S, _SD), 0)
        ej = jax.lax.broadcasted_iota(jnp.int32, (_S, _SD), 1)
        expand = (ei == ej // _D).astype(f32)
        xf = xf_ref[...]
        ms = _dot(xf * xf, blk) * (1.0 / _D)
        scale = _dot(jax.lax.rsqrt(ms + 1e-5), expand)
        h_ref[...] = xf * scale * rmsw_ref[...]

    o_ref[...] = _dot_t(h_ref[...], w_ref[...]) + b_ref[...]


def _attn_route(h_ref, lat_ref, x_ref, wq_ref, bq_ref, wk_ref, bk_ref,
                wv_ref, bv_ref, wo_ref, bo_ref, rms2_ref, rw_ref, rb_ref,
                o_ref, sh2_scr, ggt_scr, offs_vscr):
    f32 = jnp.float32
    pos = jax.lax.broadcasted_iota(jnp.int32, (_S, _D), 0).astype(f32)
    lane = jax.lax.broadcasted_iota(jnp.int32, (_S, _D), 1)
    pair = (lane % _HD) // 2
    inv_freq = jnp.exp(pair.astype(f32) * (-2.0 / _HD) * math.log(10000.0))
    ang = pos * inv_freq
    rc = jnp.cos(ang)
    rs = jnp.sin(ang)
    ssign = jnp.where(lane % 2 == 0, -rs, rs)
    ai = jax.lax.broadcasted_iota(jnp.int32, (_D, _D), 0)
    bi = jax.lax.broadcasted_iota(jnp.int32, (_D, _D), 1)
    perm = ((ai // 2 == bi // 2) & (ai != bi)).astype(f32)
    eye = (ai == bi).astype(f32)
    scale = 1.0 / math.sqrt(_HD)
    _HL = _H * _L
    kr = jax.lax.broadcasted_iota(jnp.int32, (_D, _HL), 0)
    kc = jax.lax.broadcasted_iota(jnp.int32, (_D, _HL), 1)
    kmask = (kr // _HD == kc // _L).astype(f32)
    vr = jax.lax.broadcasted_iota(jnp.int32, (_HL, _D), 0)
    vc = jax.lax.broadcasted_iota(jnp.int32, (_HL, _D), 1)
    vmask = (vr // _L == vc // _HD).astype(f32)
    sr = jax.lax.broadcasted_iota(jnp.int32, (_HL, _H), 0)
    sc2 = jax.lax.broadcasted_iota(jnp.int32, (_HL, _H), 1)
    colblk = (sr // _L == sc2).astype(f32)
    er = jax.lax.broadcasted_iota(jnp.int32, (_H, _HL), 0)
    ec = jax.lax.broadcasted_iota(jnp.int32, (_H, _HL), 1)
    rowblk = (er == ec // _L).astype(f32)

    h2s = []
    for b in range(_B):
        h = h_ref[b]
        lat = lat_ref[b]
        q = _dot_t(h, wq_ref[...]) + bq_ref[...]
        k = _dot_t(lat, wk_ref[...]) + bk_ref[...]
        v = _dot_t(lat, wv_ref[...]) + bv_ref[...]
        qr = q * rc + _dot(q, perm) * ssign
        kt = _dot_t(eye, k)
        kexp = jnp.concatenate([kt] * _H, axis=1) * kmask
        sc = _dot(qr, kexp) * scale
        mx = jnp.max(sc, axis=1, keepdims=True)
        p = jnp.exp(sc - mx)
        denom = _dot(_dot(p, colblk), rowblk)
        p = p / denom
        vexp = jnp.concatenate([v] * _H, axis=0) * vmask
        ao = _dot(p, vexp)
        attn_out = _dot_t(ao, wo_ref[...]) + bo_ref[...]
        xr = x_ref[b] + attn_out
        o_ref[pl.ds(b * _S, _S), :] = xr
        ms = jnp.mean(xr * xr, axis=1, keepdims=True)
        h2s.append(xr * jax.lax.rsqrt(ms + 1e-5) * rms2_ref[...])

    h2 = jnp.concatenate(h2s, axis=0)
    logits = _dot_t(h2, rw_ref[...]) + rb_ref[...]
    mx = jnp.max(logits, axis=1, keepdims=True)
    p = jnp.exp(logits - mx)
    probs = p / jnp.sum(p, axis=1, keepdims=True)
    iota_e = jax.lax.broadcasted_iota(jnp.int32, (_T, _E), 1)
    m1 = jnp.max(probs, axis=1, keepdims=True)
    i1 = jnp.min(jnp.where(probs == m1, iota_e, _E), axis=1, keepdims=True)
    sel1 = (iota_e == i1).astype(f32)
    pmask = jnp.where(iota_e == i1, -1.0, probs)
    m2 = jnp.max(pmask, axis=1, keepdims=True)
    i2 = jnp.min(jnp.where(pmask == m2, iota_e, _E), axis=1, keepdims=True)
    sel2 = (iota_e == i2).astype(f32)
    mask = sel1 + sel2

    ti = jax.lax.broadcasted_iota(jnp.int32, (_T, _T), 0)
    tj = jax.lax.broadcasted_iota(jnp.int32, (_T, _T), 1)
    ltri = (tj < ti).astype(f32)
    prefix = _dot(ltri, mask)
    a16 = jax.lax.broadcasted_iota(jnp.int32, (_E, 32), 0)
    b32 = jax.lax.broadcasted_iota(jnp.int32, (_E, 32), 1)
    u32 = (a16 < b32).astype(f32)
    offs32 = _dot(jnp.ones((1, _T), f32), _dot(mask, u32))
    offs_vscr[...] = offs32.astype(jnp.int32)
    offs = offs32[:, :_E]
    posm = offs + prefix
    p0 = jnp.sum(sel1 * posm, axis=1, keepdims=True)
    p1 = jnp.sum(sel2 * posm, axis=1, keepdims=True)
    one = jnp.ones((1, 1), f32)
    p0t = _dot_t(one, p0)
    p1t = _dot_t(one, p1)

    rif = jax.lax.broadcasted_iota(jnp.int32, (_RP, _T), 0).astype(f32)
    gmat = (rif == p0t).astype(f32) + (rif == p1t).astype(f32)
    sh2_scr[pl.ds(0, _RP), :] = _dot(gmat, h2)
    sh2_scr[pl.ds(_RP, _RPP - _RP), :] = jnp.zeros((_RPP - _RP, _D), f32)

    cif = jax.lax.broadcasted_iota(jnp.int32, (_T, _RP), 1).astype(f32)
    ggt_scr[...] = (m1 * (cif == p0).astype(f32)
                    + m2 * (cif == p1).astype(f32))


def _block_kernel(h_ref, lat_ref, x_ref, wq_ref, bq_ref, wk_ref, bk_ref,
                  wv_ref, bv_ref, wo_ref, bo_ref, rms2_ref, rw_ref, rb_ref,
                  w1_ref, b1_ref, ws_ref, bs_ref, w2_ref, b2_ref, o_ref,
                  sh2_scr, ggt_scr, ysc, offs_vscr, offs_sscr, sem):
    g = pl.program_id(0)

    @pl.when(g == 0)
    def _():
        _attn_route(h_ref, lat_ref, x_ref, wq_ref, bq_ref, wk_ref, bk_ref,
                    wv_ref, bv_ref, wo_ref, bo_ref, rms2_ref, rw_ref, rb_ref,
                    o_ref, sh2_scr, ggt_scr, offs_vscr)
        cp = pltpu.make_async_copy(offs_vscr, offs_sscr, sem)
        cp.start()
        cp.wait()
        ysc[...] = jnp.zeros((_RPP, _D), jnp.float32)

    @pl.when((g >= 1) & (g <= _E // _EPB))
    def _():
        bf16 = jnp.bfloat16
        for j in range(_EPB):
            e = (g - 1) * _EPB + j
            start = offs_sscr[0, e]
            end = offs_sscr[0, e + 1]
            base = (start // _MT) * _MT
            w1b = w1_ref[j].astype(bf16)
            wsb = ws_ref[j].astype(bf16)
            w2b = w2_ref[j].astype(bf16)

            def tile(st, w1b=w1b, wsb=wsb, w2b=w2b, j=j, start=start, end=end):
                rows = sh2_scr[pl.ds(st, _MT), :]
                h1 = _dot_t(rows.astype(bf16), w1b) + b1_ref[j]
                hs = jnp.maximum(_dot_t(h1.astype(bf16), wsb) + bs_ref[j],
                                 0.0)
                y = _dot_t(hs.astype(bf16), w2b) + b2_ref[j]
                rid = st + jax.lax.broadcasted_iota(jnp.int32, (_MT, _D), 0)
                valid = (rid >= start) & (rid < end)
                old = ysc[pl.ds(st, _MT), :]
                ysc[pl.ds(st, _MT), :] = jnp.where(valid, y, old)

            tile(base)
            tile(base + _MT)
            nt = (end - base + _MT - 1) // _MT

            def body(t, carry, tile=tile, base=base):
                tile(base + t * _MT)
                return carry

            jax.lax.fori_loop(2, nt, body, 0)

    @pl.when(g == _E // _EPB + 1)
    def _():
        o_ref[...] += _dot(ggt_scr[...], ysc[pl.ds(0, _RP), :])


def kernel(x, rms1_w, rms2_w, latent_W, latent_b, Wq, bq, Wk, bk, Wv, bv,
           Wo, bo, router_W, router_b, e1_W, e1_b, sw_W, sw_b, e2_W, e2_b):
    f32 = jnp.float32

    n_lat = _LD // _NBLK
    rmsw_flat = jnp.broadcast_to(rms1_w, (_S, _D)).reshape(1, _SD)
    latent, h = pl.pallas_call(
        _latent_kernel,
        grid=(n_lat,),
        in_specs=[
            pl.BlockSpec((_B, _SD), lambda i: (0, 0)),
            pl.BlockSpec((1, _SD), lambda i: (0, 0)),
            pl.BlockSpec((_NBLK, _SD), lambda i: (i, 0)),
            pl.BlockSpec((1, _NBLK), lambda i: (0, i)),
        ],
        out_specs=[
            pl.BlockSpec((_B, _NBLK), lambda i: (0, i)),
            pl.BlockSpec((_B, _SD), lambda i: (0, 0)),
        ],
        out_shape=[
            jax.ShapeDtypeStruct((_B, _LD), f32),
            jax.ShapeDtypeStruct((_B, _SD), f32),
        ],
    )(x.reshape(_B, _SD), rmsw_flat, latent_W, latent_b.reshape(1, _LD))

    row = lambda a: a.reshape(1, -1)
    cst = lambda shp: pl.BlockSpec(shp, lambda g: tuple(0 for _ in shp))
    wsel = lambda shp: pl.BlockSpec(
        shp, lambda g: (jnp.clip(g - 1, 0, _E // _EPB - 1),)
        + (0,) * (len(shp) - 1))
    out = pl.pallas_call(
        _block_kernel,
        grid=(_E // _EPB + 2,),
        in_specs=[
            cst((_B, _S, _D)), cst((_B, _L, _D)), cst((_B, _S, _D)),
            cst((_D, _D)), cst((1, _D)),
            cst((_D, _D)), cst((1, _D)),
            cst((_D, _D)), cst((1, _D)),
            cst((_D, _D)), cst((1, _D)),
            cst((1, _D)),
            cst((_E, _D)), cst((1, _E)),
            wsel((_EPB, _F, _D)), wsel((_EPB, 1, _F)),
            wsel((_EPB, _F, _F)), wsel((_EPB, 1, _F)),
            wsel((_EPB, _D, _F)), wsel((_EPB, 1, _D)),
        ],
        out_specs=pl.BlockSpec((_T, _D), lambda g: (0, 0)),
        out_shape=jax.ShapeDtypeStruct((_T, _D), f32),
        scratch_shapes=[
            pltpu.VMEM((_RPP, _D), f32),
            pltpu.VMEM((_T, _RP), f32),
            pltpu.VMEM((_RPP, _D), f32),
            pltpu.VMEM((1, 32), jnp.int32),
            pltpu.SMEM((1, 32), jnp.int32),
            pltpu.SemaphoreType.DMA,
        ],
    )(h.reshape(_B, _S, _D), latent.reshape(_B, _L, _D), x,
      Wq, row(bq), Wk, row(bk), Wv, row(bv), Wo, row(bo),
      row(rms2_w), router_W, row(router_b),
      e1_W, e1_b.reshape(_E, 1, _F), sw_W, sw_b.reshape(_E, 1, _F),
      e2_W, e2_b.reshape(_E, 1, _D))

    return out.reshape(_B, _S, _D)
